# baseline (speedup 1.0000x reference)
import functools

import jax
import jax.numpy as jnp
from jax import lax
from jax.experimental import pallas as pl
from jax.experimental.pallas import tpu as pltpu
from jax.experimental.pallas import tpu_sc as plsc

N = 10000
E = 320000
D = 128
DE = 16
H = 128

N2 = 10240
E2 = 327680
NC = 2
NS = 16
NW = NC * NS
EPW = E2 // NW
WG = 80
WS = 136
CG = 128
CS = 128


def _silu(x):
    return x * jax.nn.sigmoid(x)


def _pack_pair(even_f32, odd_f32):
    ue = jax.lax.bitcast_convert_type(even_f32, jnp.uint32)
    uo = jax.lax.bitcast_convert_type(odd_f32, jnp.uint32)
    half = jnp.uint32(0x8000)
    lo = jnp.right_shift(ue + half, jnp.uint32(16))
    hi = jnp.bitwise_and(uo + half, jnp.uint32(0xFFFF0000))
    return jax.lax.bitcast_convert_type(jnp.bitwise_or(lo, hi), jnp.float32)


def _prep_body(h_ref, cd_ref, w1h1e_ref, w1h1o_ref, w1h2e_ref, w1h2o_ref,
               b1ee_ref, b1eo_ref, a_ref, b_ref):
    hh = h_ref[...]
    ae = jnp.dot(hh, w1h1e_ref[...], preferred_element_type=jnp.float32) + b1ee_ref[...]
    ao = jnp.dot(hh, w1h1o_ref[...], preferred_element_type=jnp.float32) + b1eo_ref[...]
    be = jnp.dot(hh, w1h2e_ref[...], preferred_element_type=jnp.float32)
    bo = jnp.dot(hh, w1h2o_ref[...], preferred_element_type=jnp.float32)
    cd = cd_ref[...]
    z13 = jnp.zeros((hh.shape[0], 13), jnp.float32)
    a_ref[...] = jnp.concatenate([_pack_pair(ae, ao), cd, z13], axis=1)
    b_ref[...] = jnp.concatenate([_pack_pair(be, bo), -cd, z13], axis=1)


def _tc_prep(h_p, coord_p, w1h1e, w1h1o, w1h2e, w1h2o, b1ee, b1eo):
    nb = 2048
    grid = (N2 // nb,)
    return pl.pallas_call(
        _prep_body,
        grid=grid,
        in_specs=[
            pl.BlockSpec((nb, D), lambda i: (i, 0)),
            pl.BlockSpec((nb, 3), lambda i: (i, 0)),
            pl.BlockSpec((D, H // 2), lambda i: (0, 0)),
            pl.BlockSpec((D, H // 2), lambda i: (0, 0)),
            pl.BlockSpec((D, H // 2), lambda i: (0, 0)),
            pl.BlockSpec((D, H // 2), lambda i: (0, 0)),
            pl.BlockSpec((1, H // 2), lambda i: (0, 0)),
            pl.BlockSpec((1, H // 2), lambda i: (0, 0)),
        ],
        out_specs=[
            pl.BlockSpec((nb, WG), lambda i: (i, 0)),
            pl.BlockSpec((nb, WG), lambda i: (i, 0)),
        ],
        out_shape=[
            jax.ShapeDtypeStruct((N2, WG), jnp.float32),
            jax.ShapeDtypeStruct((N2, WG), jnp.float32),
        ],
    )(h_p, coord_p, w1h1e, w1h1o, w1h2e, w1h2o, b1ee, b1eo)


def _sc_gather(ap, bp, row2d, col2d, es):
    mesh = plsc.VectorSubcoreMesh(core_axis_name="c", subcore_axis_name="s")
    epw = es // NW
    nchunk = epw // CG

    @functools.partial(
        pl.kernel,
        mesh=mesh,
        out_type=(
            jax.ShapeDtypeStruct((es, WG), jnp.float32),
            jax.ShapeDtypeStruct((es, WG), jnp.float32),
        ),
        scratch_types=[
            pltpu.VMEM((nchunk, CG), jnp.int32),
            pltpu.VMEM((nchunk, CG), jnp.int32),
            pltpu.VMEM((2, CG, WG), jnp.float32),
            pltpu.VMEM((2, CG, WG), jnp.float32),
            pltpu.SemaphoreType.DMA,
            pltpu.SemaphoreType.DMA,
            pltpu.SemaphoreType.DMA,
            pltpu.SemaphoreType.DMA,
            pltpu.SemaphoreType.DMA,
            pltpu.SemaphoreType.DMA,
        ],
        compiler_params=pltpu.CompilerParams(use_tc_tiling_on_sc=False),
    )
    def gat(ap_hbm, bp_hbm, row_hbm, col_hbm, pa_hbm, pb_hbm,
            idxa, idxb, bufa, bufb, sga0, sga1, sgb0, sgb1, swa, swb):
        wid = lax.axis_index("s") * NC + lax.axis_index("c")
        base = wid * epw
        pltpu.sync_copy(row_hbm.at[pl.ds(wid * nchunk, nchunk)], idxa)
        pltpu.sync_copy(col_hbm.at[pl.ds(wid * nchunk, nchunk)], idxb)
        sga = (sga0, sga1)
        sgb = (sgb0, sgb1)

        def issue_gather(i, b):
            pltpu.make_async_copy(ap_hbm.at[idxa.at[i]], bufa.at[b],
                                  sga[b]).start()
            pltpu.make_async_copy(bp_hbm.at[idxb.at[i]], bufb.at[b],
                                  sgb[b]).start()

        def wait_gather(i, b):
            pltpu.make_async_copy(ap_hbm.at[idxa.at[i]], bufa.at[b],
                                  sga[b]).wait()
            pltpu.make_async_copy(bp_hbm.at[idxb.at[i]], bufb.at[b],
                                  sgb[b]).wait()

        def drain_wb(b):
            pltpu.make_async_copy(bufa.at[b], pa_hbm.at[pl.ds(base, CG)],
                                  swa).wait()
            pltpu.make_async_copy(bufb.at[b], pb_hbm.at[pl.ds(base, CG)],
                                  swb).wait()

        def start_wb(i, b):
            off = base + i * CG
            pltpu.make_async_copy(bufa.at[b], pa_hbm.at[pl.ds(off, CG)],
                                  swa).start()
            pltpu.make_async_copy(bufb.at[b], pb_hbm.at[pl.ds(off, CG)],
                                  swb).start()

        issue_gather(0, 0)

        def body(i2, carry):
            @pl.when(i2 > 0)
            def _():
                drain_wb(1)
            issue_gather(i2 * 2 + 1, 1)
            wait_gather(i2 * 2, 0)
            start_wb(i2 * 2, 0)
            drain_wb(0)

            @pl.when(i2 < nchunk // 2 - 1)
            def _():
                issue_gather(i2 * 2 + 2, 0)
            wait_gather(i2 * 2 + 1, 1)
            start_wb(i2 * 2 + 1, 1)
            return carry

        lax.fori_loop(0, nchunk // 2, body, 0)
        drain_wb(1)

    return gat(ap, bp, row2d, col2d)


def _unpack_pair(packed_f32):
    u = jax.lax.bitcast_convert_type(packed_f32, jnp.uint32)
    even = jax.lax.bitcast_convert_type(
        jnp.left_shift(u, jnp.uint32(16)), jnp.float32)
    odd = jax.lax.bitcast_convert_type(
        jnp.bitwise_and(u, jnp.uint32(0xFFFF0000)), jnp.float32)
    return even, odd


def _edge_body(pa_ref, pb_ref, ea_ref, wre_ref, wro_ref, w1ae_ref, w1ao_ref,
               w2ee_ref, w2eo_ref, b2e_ref, w1c_ref, b1c_ref, w2c_ref, out_ref):
    pa = pa_ref[...]
    pb = pb_ref[...]
    ae, ao = _unpack_pair(pa[:, :H // 2])
    be, bo = _unpack_pair(pb[:, :H // 2])
    cd = pa[:, H // 2:H // 2 + 3] + pb[:, H // 2:H // 2 + 3]
    radial = jnp.sum(cd * cd, axis=1, keepdims=True)
    ea = ea_ref[...]
    ze = ae + be + radial * wre_ref[...] + jnp.dot(
        ea, w1ae_ref[...], preferred_element_type=jnp.float32)
    zo = ao + bo + radial * wro_ref[...] + jnp.dot(
        ea, w1ao_ref[...], preferred_element_type=jnp.float32)
    ze = _silu(ze)
    zo = _silu(zo)
    ef = (jnp.dot(ze, w2ee_ref[...], preferred_element_type=jnp.float32)
          + jnp.dot(zo, w2eo_ref[...], preferred_element_type=jnp.float32)
          + b2e_ref[...])
    ef = _silu(ef)
    c1 = jnp.dot(ef, w1c_ref[...], preferred_element_type=jnp.float32) + b1c_ref[...]
    c1 = _silu(c1)
    cm = jnp.sum(c1 * w2c_ref[...], axis=1, keepdims=True)
    trans = jnp.clip(cd * cm, -100.0, 100.0)
    ones = jnp.ones((trans.shape[0], 1), jnp.float32)
    z4 = jnp.zeros((trans.shape[0], 4), jnp.float32)
    out_ref[...] = jnp.concatenate([ef, trans, ones, z4], axis=1)


def _tc_edge(pa, pb, ea_p, wre, wro, w1ae, w1ao, w2ee, w2eo, b2e, w1c, b1c,
             w2c_row):
    eb = 2048
    es = pa.shape[0]
    grid = (es // eb,)
    return pl.pallas_call(
        _edge_body,
        grid=grid,
        in_specs=[
            pl.BlockSpec((eb, WG), lambda i: (i, 0)),
            pl.BlockSpec((eb, WG), lambda i: (i, 0)),
            pl.BlockSpec((eb, DE), lambda i: (i, 0)),
            pl.BlockSpec((1, H // 2), lambda i: (0, 0)),
            pl.BlockSpec((1, H // 2), lambda i: (0, 0)),
            pl.BlockSpec((DE, H // 2), lambda i: (0, 0)),
            pl.BlockSpec((DE, H // 2), lambda i: (0, 0)),
            pl.BlockSpec((H // 2, H), lambda i: (0, 0)),
            pl.BlockSpec((H // 2, H), lambda i: (0, 0)),
            pl.BlockSpec((1, H), lambda i: (0, 0)),
            pl.BlockSpec((H, H), lambda i: (0, 0)),
            pl.BlockSpec((1, H), lambda i: (0, 0)),
            pl.BlockSpec((1, H), lambda i: (0, 0)),
        ],
        out_specs=pl.BlockSpec((eb, WS), lambda i: (i, 0)),
        out_shape=jax.ShapeDtypeStruct((es, WS), jnp.float32),
    )(pa, pb, ea_p, wre, wro, w1ae, w1ao, w2ee, w2eo, b2e, w1c, b1c, w2c_row)


def _sc_scatter(ef_ext, row_s, init_partials):
    mesh = plsc.VectorSubcoreMesh(core_axis_name="c", subcore_axis_name="s")
    es = ef_ext.shape[0]
    epw = es // NW

    @functools.partial(
        pl.kernel,
        mesh=mesh,
        out_type=jax.ShapeDtypeStruct((NC, N2, WS), jnp.float32),
        scratch_types=[
            pltpu.VMEM((CS,), jnp.int32),
            pltpu.VMEM((CS, WS), jnp.float32),
            pltpu.VMEM_SHARED((N2, WS), jnp.float32),
            pltpu.SemaphoreType.DMA,
        ],
        compiler_params=pltpu.CompilerParams(use_tc_tiling_on_sc=False),
    )
    def sca(ef_hbm, row_hbm, init_hbm, out_hbm, idxv, efv, acc, sem):
        cid = lax.axis_index("c")
        sid = lax.axis_index("s")
        wid = sid * NC + cid

        @pl.when(sid == 0)
        def _():
            pltpu.sync_copy(init_hbm.at[cid], acc)

        plsc.subcore_barrier()
        base = wid * epw

        def body(i, carry):
            off = base + i * CS
            pltpu.sync_copy(row_hbm.at[pl.ds(off, CS)], idxv)
            pltpu.sync_copy(ef_hbm.at[pl.ds(off, CS)], efv)
            pltpu.sync_copy(efv, acc.at[idxv], add=True)
            return carry

        lax.fori_loop(0, epw // CS, body, 0)
        plsc.subcore_barrier()

        @pl.when(sid == 0)
        def _():
            pltpu.sync_copy(acc, out_hbm.at[cid])

    return sca(ef_ext, row_s, init_partials)


def _node_body(p0_ref, p1_ref, h_ref, cd_ref, w1nh_ref, w1na_ref, b1n_ref,
               w2n_ref, b2n_ref, hn_ref, cn_ref):
    p = p0_ref[...] + p1_ref[...]
    agg = p[:, :H]
    num = p[:, H:H + 3]
    cnt = p[:, H + 3:H + 4]
    f = num / jnp.maximum(cnt, 1.0)
    cn_ref[...] = cd_ref[...] + f
    z = (jnp.dot(h_ref[...], w1nh_ref[...], preferred_element_type=jnp.float32)
         + jnp.dot(agg, w1na_ref[...], preferred_element_type=jnp.float32)
         + b1n_ref[...])
    z = _silu(z)
    hn_ref[...] = h_ref[...] + jnp.dot(
        z, w2n_ref[...], preferred_element_type=jnp.float32) + b2n_ref[...]


def _tc_node(p0, p1, h_p, coord_p, w1nh, w1na, b1n, w2n, b2n):
    nb = 2048
    grid = (N2 // nb,)
    return pl.pallas_call(
        _node_body,
        grid=grid,
        in_specs=[
            pl.BlockSpec((nb, WS), lambda i: (i, 0)),
            pl.BlockSpec((nb, WS), lambda i: (i, 0)),
            pl.BlockSpec((nb, D), lambda i: (i, 0)),
            pl.BlockSpec((nb, 3), lambda i: (i, 0)),
            pl.BlockSpec((D, H), lambda i: (0, 0)),
            pl.BlockSpec((H, H), lambda i: (0, 0)),
            pl.BlockSpec((1, H), lambda i: (0, 0)),
            pl.BlockSpec((H, D), lambda i: (0, 0)),
            pl.BlockSpec((1, D), lambda i: (0, 0)),
        ],
        out_specs=[
            pl.BlockSpec((nb, D), lambda i: (i, 0)),
            pl.BlockSpec((nb, 3), lambda i: (i, 0)),
        ],
        out_shape=[
            jax.ShapeDtypeStruct((N2, D), jnp.float32),
            jax.ShapeDtypeStruct((N2, 3), jnp.float32),
        ],
    )(p0, p1, h_p, coord_p, w1nh, w1na, b1n, w2n, b2n)


@jax.jit
def kernel(h, coord, edge_index, edge_attr,
           W1e, b1e, W2e, b2e, W1n, b1n, W2n, b2n, W1c, b1c, W2c):
    row = edge_index[0].astype(jnp.int32)
    col = edge_index[1].astype(jnp.int32)

    h_p = jnp.zeros((N2, D), jnp.float32).at[:N].set(h)
    coord_p = jnp.zeros((N2, 3), jnp.float32).at[:N].set(coord)
    row_p = jnp.full((E2,), N, jnp.int32).at[:E].set(row)
    col_p = jnp.zeros((E2,), jnp.int32).at[:E].set(col)
    ea_p = jnp.zeros((E2, DE), jnp.float32).at[:E].set(edge_attr)

    w1h1 = W1e[0:D]
    w1h2 = W1e[D:2 * D]
    wr = W1e[2 * D:2 * D + 1]
    w1a = W1e[2 * D + 1:]
    w1h1e, w1h1o = w1h1[:, 0::2], w1h1[:, 1::2]
    w1h2e, w1h2o = w1h2[:, 0::2], w1h2[:, 1::2]
    wre, wro = wr[:, 0::2], wr[:, 1::2]
    w1ae, w1ao = w1a[:, 0::2], w1a[:, 1::2]
    b1ee = b1e[0::2].reshape(1, H // 2)
    b1eo = b1e[1::2].reshape(1, H // 2)
    w2ee, w2eo = W2e[0::2, :], W2e[1::2, :]
    b2e_r = b2e.reshape(1, H)
    b1c_r = b1c.reshape(1, H)
    w2c_row = W2c.reshape(1, H)
    w1nh = W1n[:D]
    w1na = W1n[D:]
    b1n_r = b1n.reshape(1, H)
    b2n_r = b2n.reshape(1, D)

    ap, bp = _tc_prep(h_p, coord_p, w1h1e, w1h1o, w1h2e, w1h2o, b1ee, b1eo)

    nslice = 2
    es = E2 // nslice
    row2d = row_p.reshape(nslice, es)
    col2d = col_p.reshape(nslice, es)
    partials = jnp.zeros((NC, N2, WS), jnp.float32)
    for s in range(nslice):
        row_s = row2d[s]
        col_s = col2d[s]
        pa, pb = _sc_gather(ap, bp, row_s.reshape(-1, CG),
                            col_s.reshape(-1, CG), es)
        ef_ext = _tc_edge(pa, pb,
                          lax.dynamic_slice_in_dim(ea_p, s * es, es),
                          wre, wro, w1ae, w1ao, w2ee, w2eo, b2e_r,
                          W1c, b1c_r, w2c_row)
        partials = _sc_scatter(ef_ext, row_s, partials)

    hn, cn = _tc_node(partials[0], partials[1], h_p, coord_p,
                      w1nh, w1na, b1n_r, W2n, b2n_r)
    return hn[:N], cn[:N]


# speedup vs baseline: 1.0052x; 1.0052x over previous
import functools

import jax
import jax.numpy as jnp
from jax import lax
from jax.experimental import pallas as pl
from jax.experimental.pallas import tpu as pltpu
from jax.experimental.pallas import tpu_sc as plsc

N = 10000
E = 320000
D = 128
DE = 16
H = 128

N2 = 10240
E2 = 327680
NC = 2
NS = 16
NW = NC * NS
EPW = E2 // NW
WG = 80
WS = 136
CG = 128
CS = 128


def _silu(x):
    return x * jax.nn.sigmoid(x)


def _pack_pair(even_f32, odd_f32):
    ue = jax.lax.bitcast_convert_type(even_f32, jnp.uint32)
    uo = jax.lax.bitcast_convert_type(odd_f32, jnp.uint32)
    half = jnp.uint32(0x8000)
    lo = jnp.right_shift(ue + half, jnp.uint32(16))
    hi = jnp.bitwise_and(uo + half, jnp.uint32(0xFFFF0000))
    return jax.lax.bitcast_convert_type(jnp.bitwise_or(lo, hi), jnp.float32)


def _prep_body(h_ref, cd_ref, w1h1e_ref, w1h1o_ref, w1h2e_ref, w1h2o_ref,
               b1ee_ref, b1eo_ref, a_ref, b_ref):
    hh = h_ref[...]
    ae = jnp.dot(hh, w1h1e_ref[...], preferred_element_type=jnp.float32) + b1ee_ref[...]
    ao = jnp.dot(hh, w1h1o_ref[...], preferred_element_type=jnp.float32) + b1eo_ref[...]
    be = jnp.dot(hh, w1h2e_ref[...], preferred_element_type=jnp.float32)
    bo = jnp.dot(hh, w1h2o_ref[...], preferred_element_type=jnp.float32)
    cd = cd_ref[...]
    z13 = jnp.zeros((hh.shape[0], 13), jnp.float32)
    a_ref[...] = jnp.concatenate([_pack_pair(ae, ao), cd, z13], axis=1)
    b_ref[...] = jnp.concatenate([_pack_pair(be, bo), -cd, z13], axis=1)


def _tc_prep(h_p, coord_p, w1h1e, w1h1o, w1h2e, w1h2o, b1ee, b1eo):
    nb = 2048
    grid = (N2 // nb,)
    return pl.pallas_call(
        _prep_body,
        grid=grid,
        in_specs=[
            pl.BlockSpec((nb, D), lambda i: (i, 0)),
            pl.BlockSpec((nb, 3), lambda i: (i, 0)),
            pl.BlockSpec((D, H // 2), lambda i: (0, 0)),
            pl.BlockSpec((D, H // 2), lambda i: (0, 0)),
            pl.BlockSpec((D, H // 2), lambda i: (0, 0)),
            pl.BlockSpec((D, H // 2), lambda i: (0, 0)),
            pl.BlockSpec((1, H // 2), lambda i: (0, 0)),
            pl.BlockSpec((1, H // 2), lambda i: (0, 0)),
        ],
        out_specs=[
            pl.BlockSpec((nb, WG), lambda i: (i, 0)),
            pl.BlockSpec((nb, WG), lambda i: (i, 0)),
        ],
        out_shape=[
            jax.ShapeDtypeStruct((N2, WG), jnp.float32),
            jax.ShapeDtypeStruct((N2, WG), jnp.float32),
        ],
    )(h_p, coord_p, w1h1e, w1h1o, w1h2e, w1h2o, b1ee, b1eo)


def _sc_gather(ap, bp, row2d, col2d, es):
    mesh = plsc.VectorSubcoreMesh(core_axis_name="c", subcore_axis_name="s")
    epw = es // NW
    nchunk = epw // CG

    @functools.partial(
        pl.kernel,
        mesh=mesh,
        out_type=(
            jax.ShapeDtypeStruct((es, WG), jnp.float32),
            jax.ShapeDtypeStruct((es, WG), jnp.float32),
        ),
        scratch_types=[
            pltpu.VMEM((nchunk, CG), jnp.int32),
            pltpu.VMEM((nchunk, CG), jnp.int32),
            pltpu.VMEM((2, CG, WG), jnp.float32),
            pltpu.VMEM((2, CG, WG), jnp.float32),
            pltpu.SemaphoreType.DMA,
            pltpu.SemaphoreType.DMA,
            pltpu.SemaphoreType.DMA,
            pltpu.SemaphoreType.DMA,
            pltpu.SemaphoreType.DMA,
            pltpu.SemaphoreType.DMA,
        ],
        compiler_params=pltpu.CompilerParams(use_tc_tiling_on_sc=False),
    )
    def gat(ap_hbm, bp_hbm, row_hbm, col_hbm, pa_hbm, pb_hbm,
            idxa, idxb, bufa, bufb, sga0, sga1, sgb0, sgb1, swa, swb):
        wid = lax.axis_index("s") * NC + lax.axis_index("c")
        base = wid * epw
        pltpu.sync_copy(row_hbm.at[pl.ds(wid * nchunk, nchunk)], idxa)
        pltpu.sync_copy(col_hbm.at[pl.ds(wid * nchunk, nchunk)], idxb)
        sga = (sga0, sga1)
        sgb = (sgb0, sgb1)

        def issue_gather(i, b):
            pltpu.make_async_copy(ap_hbm.at[idxa.at[i]], bufa.at[b],
                                  sga[b]).start()
            pltpu.make_async_copy(bp_hbm.at[idxb.at[i]], bufb.at[b],
                                  sgb[b]).start()

        def wait_gather(i, b):
            pltpu.make_async_copy(ap_hbm.at[idxa.at[i]], bufa.at[b],
                                  sga[b]).wait()
            pltpu.make_async_copy(bp_hbm.at[idxb.at[i]], bufb.at[b],
                                  sgb[b]).wait()

        def drain_wb(b):
            pltpu.make_async_copy(bufa.at[b], pa_hbm.at[pl.ds(base, CG)],
                                  swa).wait()
            pltpu.make_async_copy(bufb.at[b], pb_hbm.at[pl.ds(base, CG)],
                                  swb).wait()

        def start_wb(i, b):
            off = base + i * CG
            pltpu.make_async_copy(bufa.at[b], pa_hbm.at[pl.ds(off, CG)],
                                  swa).start()
            pltpu.make_async_copy(bufb.at[b], pb_hbm.at[pl.ds(off, CG)],
                                  swb).start()

        issue_gather(0, 0)

        def body(i2, carry):
            @pl.when(i2 > 0)
            def _():
                drain_wb(1)
            issue_gather(i2 * 2 + 1, 1)
            wait_gather(i2 * 2, 0)
            start_wb(i2 * 2, 0)
            drain_wb(0)

            @pl.when(i2 < nchunk // 2 - 1)
            def _():
                issue_gather(i2 * 2 + 2, 0)
            wait_gather(i2 * 2 + 1, 1)
            start_wb(i2 * 2 + 1, 1)
            return carry

        lax.fori_loop(0, nchunk // 2, body, 0)
        drain_wb(1)

    return gat(ap, bp, row2d, col2d)


def _unpack_pair(packed_f32):
    u = jax.lax.bitcast_convert_type(packed_f32, jnp.uint32)
    even = jax.lax.bitcast_convert_type(
        jnp.left_shift(u, jnp.uint32(16)), jnp.float32)
    odd = jax.lax.bitcast_convert_type(
        jnp.bitwise_and(u, jnp.uint32(0xFFFF0000)), jnp.float32)
    return even, odd


def _edge_body(pa_ref, pb_ref, ea_ref, wre_ref, wro_ref, w1ae_ref, w1ao_ref,
               w2ee_ref, w2eo_ref, b2e_ref, w1c_ref, b1c_ref, w2c_ref, out_ref):
    pa = pa_ref[...]
    pb = pb_ref[...]
    ae, ao = _unpack_pair(pa[:, :H // 2])
    be, bo = _unpack_pair(pb[:, :H // 2])
    cd = pa[:, H // 2:H // 2 + 3] + pb[:, H // 2:H // 2 + 3]
    radial = jnp.sum(cd * cd, axis=1, keepdims=True)
    ea = ea_ref[...]
    ze = ae + be + radial * wre_ref[...] + jnp.dot(
        ea, w1ae_ref[...], preferred_element_type=jnp.float32)
    zo = ao + bo + radial * wro_ref[...] + jnp.dot(
        ea, w1ao_ref[...], preferred_element_type=jnp.float32)
    ze = _silu(ze)
    zo = _silu(zo)
    ef = (jnp.dot(ze, w2ee_ref[...], preferred_element_type=jnp.float32)
          + jnp.dot(zo, w2eo_ref[...], preferred_element_type=jnp.float32)
          + b2e_ref[...])
    ef = _silu(ef)
    c1 = jnp.dot(ef, w1c_ref[...], preferred_element_type=jnp.float32) + b1c_ref[...]
    c1 = _silu(c1)
    cm = jnp.sum(c1 * w2c_ref[...], axis=1, keepdims=True)
    trans = jnp.clip(cd * cm, -100.0, 100.0)
    ones = jnp.ones((trans.shape[0], 1), jnp.float32)
    z4 = jnp.zeros((trans.shape[0], 4), jnp.float32)
    out_ref[...] = jnp.concatenate([ef, trans, ones, z4], axis=1)


def _tc_edge(pa, pb, ea_p, wre, wro, w1ae, w1ao, w2ee, w2eo, b2e, w1c, b1c,
             w2c_row):
    eb = 2048
    es = pa.shape[0]
    grid = (es // eb,)
    return pl.pallas_call(
        _edge_body,
        grid=grid,
        in_specs=[
            pl.BlockSpec((eb, WG), lambda i: (i, 0)),
            pl.BlockSpec((eb, WG), lambda i: (i, 0)),
            pl.BlockSpec((eb, DE), lambda i: (i, 0)),
            pl.BlockSpec((1, H // 2), lambda i: (0, 0)),
            pl.BlockSpec((1, H // 2), lambda i: (0, 0)),
            pl.BlockSpec((DE, H // 2), lambda i: (0, 0)),
            pl.BlockSpec((DE, H // 2), lambda i: (0, 0)),
            pl.BlockSpec((H // 2, H), lambda i: (0, 0)),
            pl.BlockSpec((H // 2, H), lambda i: (0, 0)),
            pl.BlockSpec((1, H), lambda i: (0, 0)),
            pl.BlockSpec((H, H), lambda i: (0, 0)),
            pl.BlockSpec((1, H), lambda i: (0, 0)),
            pl.BlockSpec((1, H), lambda i: (0, 0)),
        ],
        out_specs=pl.BlockSpec((eb, WS), lambda i: (i, 0)),
        out_shape=jax.ShapeDtypeStruct((es, WS), jnp.float32),
    )(pa, pb, ea_p, wre, wro, w1ae, w1ao, w2ee, w2eo, b2e, w1c, b1c, w2c_row)


def _sc_scatter(ef_ext, row_s, init_partials):
    mesh = plsc.VectorSubcoreMesh(core_axis_name="c", subcore_axis_name="s")
    es = ef_ext.shape[0]
    epw = es // NW

    @functools.partial(
        pl.kernel,
        mesh=mesh,
        out_type=jax.ShapeDtypeStruct((NC, N2, WS), jnp.float32),
        scratch_types=[
            pltpu.VMEM((CS,), jnp.int32),
            pltpu.VMEM((CS, WS), jnp.float32),
            pltpu.VMEM_SHARED((N2, WS), jnp.float32),
            pltpu.SemaphoreType.DMA,
        ],
        compiler_params=pltpu.CompilerParams(use_tc_tiling_on_sc=False),
    )
    def sca(ef_hbm, row_hbm, init_hbm, out_hbm, idxv, efv, acc, sem):
        cid = lax.axis_index("c")
        sid = lax.axis_index("s")
        wid = sid * NC + cid

        @pl.when(sid == 0)
        def _():
            pltpu.sync_copy(init_hbm.at[cid], acc)

        plsc.subcore_barrier()
        base = wid * epw

        def body(i, carry):
            off = base + i * CS
            pltpu.sync_copy(row_hbm.at[pl.ds(off, CS)], idxv)
            pltpu.sync_copy(ef_hbm.at[pl.ds(off, CS)], efv)
            pltpu.sync_copy(efv, acc.at[idxv], add=True)
            return carry

        lax.fori_loop(0, epw // CS, body, 0)
        plsc.subcore_barrier()

        @pl.when(sid == 0)
        def _():
            pltpu.sync_copy(acc, out_hbm.at[cid])

    return sca(ef_ext, row_s, init_partials)


def _node_body(p0_ref, p1_ref, h_ref, cd_ref, w1nh_ref, w1na_ref, b1n_ref,
               w2n_ref, b2n_ref, hn_ref, cn_ref):
    p = p0_ref[...] + p1_ref[...]
    agg = p[:, :H]
    num = p[:, H:H + 3]
    cnt = p[:, H + 3:H + 4]
    f = num / jnp.maximum(cnt, 1.0)
    cn_ref[...] = cd_ref[...] + f
    z = (jnp.dot(h_ref[...], w1nh_ref[...], preferred_element_type=jnp.float32)
         + jnp.dot(agg, w1na_ref[...], preferred_element_type=jnp.float32)
         + b1n_ref[...])
    z = _silu(z)
    hn_ref[...] = h_ref[...] + jnp.dot(
        z, w2n_ref[...], preferred_element_type=jnp.float32) + b2n_ref[...]


def _tc_node(p0, p1, h_p, coord_p, w1nh, w1na, b1n, w2n, b2n):
    nb = 2048
    grid = (N2 // nb,)
    return pl.pallas_call(
        _node_body,
        grid=grid,
        in_specs=[
            pl.BlockSpec((nb, WS), lambda i: (i, 0)),
            pl.BlockSpec((nb, WS), lambda i: (i, 0)),
            pl.BlockSpec((nb, D), lambda i: (i, 0)),
            pl.BlockSpec((nb, 3), lambda i: (i, 0)),
            pl.BlockSpec((D, H), lambda i: (0, 0)),
            pl.BlockSpec((H, H), lambda i: (0, 0)),
            pl.BlockSpec((1, H), lambda i: (0, 0)),
            pl.BlockSpec((H, D), lambda i: (0, 0)),
            pl.BlockSpec((1, D), lambda i: (0, 0)),
        ],
        out_specs=[
            pl.BlockSpec((nb, D), lambda i: (i, 0)),
            pl.BlockSpec((nb, 3), lambda i: (i, 0)),
        ],
        out_shape=[
            jax.ShapeDtypeStruct((N2, D), jnp.float32),
            jax.ShapeDtypeStruct((N2, 3), jnp.float32),
        ],
    )(p0, p1, h_p, coord_p, w1nh, w1na, b1n, w2n, b2n)


@jax.jit
def kernel(h, coord, edge_index, edge_attr,
           W1e, b1e, W2e, b2e, W1n, b1n, W2n, b2n, W1c, b1c, W2c):
    row = edge_index[0].astype(jnp.int32)
    col = edge_index[1].astype(jnp.int32)

    h_p = jnp.zeros((N2, D), jnp.float32).at[:N].set(h)
    coord_p = jnp.zeros((N2, 3), jnp.float32).at[:N].set(coord)
    row_p = jnp.full((E2,), N, jnp.int32).at[:E].set(row)
    col_p = jnp.zeros((E2,), jnp.int32).at[:E].set(col)
    ea_p = jnp.zeros((E2, DE), jnp.float32).at[:E].set(edge_attr)

    w1h1 = W1e[0:D]
    w1h2 = W1e[D:2 * D]
    wr = W1e[2 * D:2 * D + 1]
    w1a = W1e[2 * D + 1:]
    w1h1e, w1h1o = w1h1[:, 0::2], w1h1[:, 1::2]
    w1h2e, w1h2o = w1h2[:, 0::2], w1h2[:, 1::2]
    wre, wro = wr[:, 0::2], wr[:, 1::2]
    w1ae, w1ao = w1a[:, 0::2], w1a[:, 1::2]
    b1ee = b1e[0::2].reshape(1, H // 2)
    b1eo = b1e[1::2].reshape(1, H // 2)
    w2ee, w2eo = W2e[0::2, :], W2e[1::2, :]
    b2e_r = b2e.reshape(1, H)
    b1c_r = b1c.reshape(1, H)
    w2c_row = W2c.reshape(1, H)
    w1nh = W1n[:D]
    w1na = W1n[D:]
    b1n_r = b1n.reshape(1, H)
    b2n_r = b2n.reshape(1, D)

    ap, bp = _tc_prep(h_p, coord_p, w1h1e, w1h1o, w1h2e, w1h2o, b1ee, b1eo)

    nslice = 8
    es = E2 // nslice
    row2d = row_p.reshape(nslice, es)
    col2d = col_p.reshape(nslice, es)
    partials = jnp.zeros((NC, N2, WS), jnp.float32)
    for s in range(nslice):
        row_s = row2d[s]
        col_s = col2d[s]
        pa, pb = _sc_gather(ap, bp, row_s.reshape(-1, CG),
                            col_s.reshape(-1, CG), es)
        ef_ext = _tc_edge(pa, pb,
                          lax.dynamic_slice_in_dim(ea_p, s * es, es),
                          wre, wro, w1ae, w1ao, w2ee, w2eo, b2e_r,
                          W1c, b1c_r, w2c_row)
        partials = _sc_scatter(ef_ext, row_s, partials)

    hn, cn = _tc_node(partials[0], partials[1], h_p, coord_p,
                      w1nh, w1na, b1n_r, W2n, b2n_r)
    return hn[:N], cn[:N]


# speedup vs baseline: 1.0309x; 1.0256x over previous
import functools

import jax
import jax.numpy as jnp
from jax import lax
from jax.experimental import pallas as pl
from jax.experimental.pallas import tpu as pltpu
from jax.experimental.pallas import tpu_sc as plsc

N = 10000
E = 320000
D = 128
DE = 16
H = 128

N2 = 10240
E2 = 327680
NC = 2
NS = 16
NW = NC * NS
EPW = E2 // NW
WG = 80
WS = 136
CG = 128
CS = 128


def _silu(x):
    return x * jax.nn.sigmoid(x)


def _pack_pair(even_f32, odd_f32):
    ue = jax.lax.bitcast_convert_type(even_f32, jnp.uint32)
    uo = jax.lax.bitcast_convert_type(odd_f32, jnp.uint32)
    half = jnp.uint32(0x8000)
    lo = jnp.right_shift(ue + half, jnp.uint32(16))
    hi = jnp.bitwise_and(uo + half, jnp.uint32(0xFFFF0000))
    return jax.lax.bitcast_convert_type(jnp.bitwise_or(lo, hi), jnp.float32)


def _prep_body(h_ref, cd_ref, w1h1e_ref, w1h1o_ref, w1h2e_ref, w1h2o_ref,
               b1ee_ref, b1eo_ref, a_ref, b_ref):
    hh = h_ref[...]
    ae = jnp.dot(hh, w1h1e_ref[...], preferred_element_type=jnp.float32) + b1ee_ref[...]
    ao = jnp.dot(hh, w1h1o_ref[...], preferred_element_type=jnp.float32) + b1eo_ref[...]
    be = jnp.dot(hh, w1h2e_ref[...], preferred_element_type=jnp.float32)
    bo = jnp.dot(hh, w1h2o_ref[...], preferred_element_type=jnp.float32)
    cd = cd_ref[...]
    z13 = jnp.zeros((hh.shape[0], 13), jnp.float32)
    a_ref[...] = jnp.concatenate([_pack_pair(ae, ao), cd, z13], axis=1)
    b_ref[...] = jnp.concatenate([_pack_pair(be, bo), -cd, z13], axis=1)


def _tc_prep(h_p, coord_p, w1h1e, w1h1o, w1h2e, w1h2o, b1ee, b1eo):
    nb = 2048
    grid = (N2 // nb,)
    return pl.pallas_call(
        _prep_body,
        grid=grid,
        in_specs=[
            pl.BlockSpec((nb, D), lambda i: (i, 0)),
            pl.BlockSpec((nb, 3), lambda i: (i, 0)),
            pl.BlockSpec((D, H // 2), lambda i: (0, 0)),
            pl.BlockSpec((D, H // 2), lambda i: (0, 0)),
            pl.BlockSpec((D, H // 2), lambda i: (0, 0)),
            pl.BlockSpec((D, H // 2), lambda i: (0, 0)),
            pl.BlockSpec((1, H // 2), lambda i: (0, 0)),
            pl.BlockSpec((1, H // 2), lambda i: (0, 0)),
        ],
        out_specs=[
            pl.BlockSpec((nb, WG), lambda i: (i, 0)),
            pl.BlockSpec((nb, WG), lambda i: (i, 0)),
        ],
        out_shape=[
            jax.ShapeDtypeStruct((N2, WG), jnp.float32),
            jax.ShapeDtypeStruct((N2, WG), jnp.float32),
        ],
    )(h_p, coord_p, w1h1e, w1h1o, w1h2e, w1h2o, b1ee, b1eo)


def _sc_gather(ap, bp, row2d, col2d, es):
    mesh = plsc.VectorSubcoreMesh(core_axis_name="c", subcore_axis_name="s")
    epw = es // NW
    nchunk = epw // CG

    @functools.partial(
        pl.kernel,
        mesh=mesh,
        out_type=(
            jax.ShapeDtypeStruct((es, WG), jnp.float32),
            jax.ShapeDtypeStruct((es, WG), jnp.float32),
        ),
        scratch_types=[
            pltpu.VMEM((nchunk, CG), jnp.int32),
            pltpu.VMEM((nchunk, CG), jnp.int32),
            pltpu.VMEM((2, CG, WG), jnp.float32),
            pltpu.VMEM((2, CG, WG), jnp.float32),
            pltpu.SemaphoreType.DMA,
            pltpu.SemaphoreType.DMA,
            pltpu.SemaphoreType.DMA,
            pltpu.SemaphoreType.DMA,
            pltpu.SemaphoreType.DMA,
            pltpu.SemaphoreType.DMA,
        ],
        compiler_params=pltpu.CompilerParams(use_tc_tiling_on_sc=False),
    )
    def gat(ap_hbm, bp_hbm, row_hbm, col_hbm, pa_hbm, pb_hbm,
            idxa, idxb, bufa, bufb, sga0, sga1, sgb0, sgb1, swa, swb):
        wid = lax.axis_index("s") * NC + lax.axis_index("c")
        base = wid * epw
        pltpu.sync_copy(row_hbm.at[pl.ds(wid * nchunk, nchunk)], idxa)
        pltpu.sync_copy(col_hbm.at[pl.ds(wid * nchunk, nchunk)], idxb)
        sga = (sga0, sga1)
        sgb = (sgb0, sgb1)

        def issue_gather(i, b):
            pltpu.make_async_copy(ap_hbm.at[idxa.at[i]], bufa.at[b],
                                  sga[b]).start()
            pltpu.make_async_copy(bp_hbm.at[idxb.at[i]], bufb.at[b],
                                  sgb[b]).start()

        def wait_gather(i, b):
            pltpu.make_async_copy(ap_hbm.at[idxa.at[i]], bufa.at[b],
                                  sga[b]).wait()
            pltpu.make_async_copy(bp_hbm.at[idxb.at[i]], bufb.at[b],
                                  sgb[b]).wait()

        def drain_wb(b):
            pltpu.make_async_copy(bufa.at[b], pa_hbm.at[pl.ds(base, CG)],
                                  swa).wait()
            pltpu.make_async_copy(bufb.at[b], pb_hbm.at[pl.ds(base, CG)],
                                  swb).wait()

        def start_wb(i, b):
            off = base + i * CG
            pltpu.make_async_copy(bufa.at[b], pa_hbm.at[pl.ds(off, CG)],
                                  swa).start()
            pltpu.make_async_copy(bufb.at[b], pb_hbm.at[pl.ds(off, CG)],
                                  swb).start()

        issue_gather(0, 0)

        def body(i2, carry):
            @pl.when(i2 > 0)
            def _():
                drain_wb(1)
            issue_gather(i2 * 2 + 1, 1)
            wait_gather(i2 * 2, 0)
            start_wb(i2 * 2, 0)
            drain_wb(0)

            @pl.when(i2 < nchunk // 2 - 1)
            def _():
                issue_gather(i2 * 2 + 2, 0)
            wait_gather(i2 * 2 + 1, 1)
            start_wb(i2 * 2 + 1, 1)
            return carry

        lax.fori_loop(0, nchunk // 2, body, 0)
        drain_wb(1)

    return gat(ap, bp, row2d, col2d)


def _unpack_pair(packed_f32):
    u = jax.lax.bitcast_convert_type(packed_f32, jnp.uint32)
    even = jax.lax.bitcast_convert_type(
        jnp.left_shift(u, jnp.uint32(16)), jnp.float32)
    odd = jax.lax.bitcast_convert_type(
        jnp.bitwise_and(u, jnp.uint32(0xFFFF0000)), jnp.float32)
    return even, odd


def _edge_body(pa_ref, pb_ref, ea_ref, wre_ref, wro_ref, w1ae_ref, w1ao_ref,
               w2ee_ref, w2eo_ref, b2e_ref, w1c_ref, b1c_ref, w2c_ref, out_ref):
    pa = pa_ref[...]
    pb = pb_ref[...]
    ae, ao = _unpack_pair(pa[:, :H // 2])
    be, bo = _unpack_pair(pb[:, :H // 2])
    cd = pa[:, H // 2:H // 2 + 3] + pb[:, H // 2:H // 2 + 3]
    radial = jnp.sum(cd * cd, axis=1, keepdims=True)
    ea = ea_ref[...]
    ze = ae + be + radial * wre_ref[...] + jnp.dot(
        ea, w1ae_ref[...], preferred_element_type=jnp.float32)
    zo = ao + bo + radial * wro_ref[...] + jnp.dot(
        ea, w1ao_ref[...], preferred_element_type=jnp.float32)
    ze = _silu(ze)
    zo = _silu(zo)
    ef = (jnp.dot(ze, w2ee_ref[...], preferred_element_type=jnp.float32)
          + jnp.dot(zo, w2eo_ref[...], preferred_element_type=jnp.float32)
          + b2e_ref[...])
    ef = _silu(ef)
    c1 = jnp.dot(ef, w1c_ref[...], preferred_element_type=jnp.float32) + b1c_ref[...]
    c1 = _silu(c1)
    cm = jnp.sum(c1 * w2c_ref[...], axis=1, keepdims=True)
    trans = jnp.clip(cd * cm, -100.0, 100.0)
    ones = jnp.ones((trans.shape[0], 1), jnp.float32)
    z4 = jnp.zeros((trans.shape[0], 4), jnp.float32)
    out_ref[...] = jnp.concatenate([ef, trans, ones, z4], axis=1)


def _tc_edge(pa, pb, ea_p, wre, wro, w1ae, w1ao, w2ee, w2eo, b2e, w1c, b1c,
             w2c_row):
    eb = 2048
    es = pa.shape[0]
    grid = (es // eb,)
    return pl.pallas_call(
        _edge_body,
        grid=grid,
        in_specs=[
            pl.BlockSpec((eb, WG), lambda i: (i, 0)),
            pl.BlockSpec((eb, WG), lambda i: (i, 0)),
            pl.BlockSpec((eb, DE), lambda i: (i, 0)),
            pl.BlockSpec((1, H // 2), lambda i: (0, 0)),
            pl.BlockSpec((1, H // 2), lambda i: (0, 0)),
            pl.BlockSpec((DE, H // 2), lambda i: (0, 0)),
            pl.BlockSpec((DE, H // 2), lambda i: (0, 0)),
            pl.BlockSpec((H // 2, H), lambda i: (0, 0)),
            pl.BlockSpec((H // 2, H), lambda i: (0, 0)),
            pl.BlockSpec((1, H), lambda i: (0, 0)),
            pl.BlockSpec((H, H), lambda i: (0, 0)),
            pl.BlockSpec((1, H), lambda i: (0, 0)),
            pl.BlockSpec((1, H), lambda i: (0, 0)),
        ],
        out_specs=pl.BlockSpec((eb, WS), lambda i: (i, 0)),
        out_shape=jax.ShapeDtypeStruct((es, WS), jnp.float32),
    )(pa, pb, ea_p, wre, wro, w1ae, w1ao, w2ee, w2eo, b2e, w1c, b1c, w2c_row)


def _sc_scatter(ef_ext, row2d, init_partials):
    mesh = plsc.VectorSubcoreMesh(core_axis_name="c", subcore_axis_name="s")
    es = ef_ext.shape[0]
    epw = es // NW
    nchunk = epw // CS

    @functools.partial(
        pl.kernel,
        mesh=mesh,
        out_type=jax.ShapeDtypeStruct((NC, N2, WS), jnp.float32),
        scratch_types=[
            pltpu.VMEM((nchunk, CS), jnp.int32),
            pltpu.VMEM((2, CS, WS), jnp.float32),
            pltpu.VMEM_SHARED((N2, WS), jnp.float32),
            pltpu.SemaphoreType.DMA,
            pltpu.SemaphoreType.DMA,
        ],
        compiler_params=pltpu.CompilerParams(use_tc_tiling_on_sc=False),
    )
    def sca(ef_hbm, row_hbm, init_hbm, out_hbm, idx2d, efv, acc, sl0, sl1):
        cid = lax.axis_index("c")
        sid = lax.axis_index("s")
        wid = sid * NC + cid

        @pl.when(sid == 0)
        def _():
            pltpu.sync_copy(init_hbm.at[cid], acc)

        base = wid * epw
        pltpu.sync_copy(row_hbm.at[pl.ds(wid * nchunk, nchunk)], idx2d)
        plsc.subcore_barrier()
        sl = (sl0, sl1)

        def start_load(i, b):
            pltpu.make_async_copy(ef_hbm.at[pl.ds(base + i * CS, CS)],
                                  efv.at[b], sl[b]).start()

        def wait_load(b):
            pltpu.make_async_copy(ef_hbm.at[pl.ds(base, CS)], efv.at[b],
                                  sl[b]).wait()

        start_load(0, 0)

        def body(i2, carry):
            for b in range(2):
                i = i2 * 2 + b
                if b == 0:
                    start_load(i + 1, 1)
                else:
                    @pl.when(i2 < nchunk // 2 - 1)
                    def _():
                        start_load(i + 1, 0)
                wait_load(b)
                pltpu.sync_copy(efv.at[b], acc.at[idx2d.at[i]], add=True)
            return carry

        lax.fori_loop(0, nchunk // 2, body, 0)
        plsc.subcore_barrier()

        @pl.when(sid == 0)
        def _():
            pltpu.sync_copy(acc, out_hbm.at[cid])

    return sca(ef_ext, row2d, init_partials)


def _node_body(p0_ref, p1_ref, h_ref, cd_ref, w1nh_ref, w1na_ref, b1n_ref,
               w2n_ref, b2n_ref, hn_ref, cn_ref):
    p = p0_ref[...] + p1_ref[...]
    agg = p[:, :H]
    num = p[:, H:H + 3]
    cnt = p[:, H + 3:H + 4]
    f = num / jnp.maximum(cnt, 1.0)
    cn_ref[...] = cd_ref[...] + f
    z = (jnp.dot(h_ref[...], w1nh_ref[...], preferred_element_type=jnp.float32)
         + jnp.dot(agg, w1na_ref[...], preferred_element_type=jnp.float32)
         + b1n_ref[...])
    z = _silu(z)
    hn_ref[...] = h_ref[...] + jnp.dot(
        z, w2n_ref[...], preferred_element_type=jnp.float32) + b2n_ref[...]


def _tc_node(p0, p1, h_p, coord_p, w1nh, w1na, b1n, w2n, b2n):
    nb = 2048
    grid = (N2 // nb,)
    return pl.pallas_call(
        _node_body,
        grid=grid,
        in_specs=[
            pl.BlockSpec((nb, WS), lambda i: (i, 0)),
            pl.BlockSpec((nb, WS), lambda i: (i, 0)),
            pl.BlockSpec((nb, D), lambda i: (i, 0)),
            pl.BlockSpec((nb, 3), lambda i: (i, 0)),
            pl.BlockSpec((D, H), lambda i: (0, 0)),
            pl.BlockSpec((H, H), lambda i: (0, 0)),
            pl.BlockSpec((1, H), lambda i: (0, 0)),
            pl.BlockSpec((H, D), lambda i: (0, 0)),
            pl.BlockSpec((1, D), lambda i: (0, 0)),
        ],
        out_specs=[
            pl.BlockSpec((nb, D), lambda i: (i, 0)),
            pl.BlockSpec((nb, 3), lambda i: (i, 0)),
        ],
        out_shape=[
            jax.ShapeDtypeStruct((N2, D), jnp.float32),
            jax.ShapeDtypeStruct((N2, 3), jnp.float32),
        ],
    )(p0, p1, h_p, coord_p, w1nh, w1na, b1n, w2n, b2n)


@jax.jit
def kernel(h, coord, edge_index, edge_attr,
           W1e, b1e, W2e, b2e, W1n, b1n, W2n, b2n, W1c, b1c, W2c):
    row = edge_index[0].astype(jnp.int32)
    col = edge_index[1].astype(jnp.int32)

    h_p = jnp.zeros((N2, D), jnp.float32).at[:N].set(h)
    coord_p = jnp.zeros((N2, 3), jnp.float32).at[:N].set(coord)
    row_p = jnp.full((E2,), N, jnp.int32).at[:E].set(row)
    col_p = jnp.zeros((E2,), jnp.int32).at[:E].set(col)
    ea_p = jnp.zeros((E2, DE), jnp.float32).at[:E].set(edge_attr)

    w1h1 = W1e[0:D]
    w1h2 = W1e[D:2 * D]
    wr = W1e[2 * D:2 * D + 1]
    w1a = W1e[2 * D + 1:]
    w1h1e, w1h1o = w1h1[:, 0::2], w1h1[:, 1::2]
    w1h2e, w1h2o = w1h2[:, 0::2], w1h2[:, 1::2]
    wre, wro = wr[:, 0::2], wr[:, 1::2]
    w1ae, w1ao = w1a[:, 0::2], w1a[:, 1::2]
    b1ee = b1e[0::2].reshape(1, H // 2)
    b1eo = b1e[1::2].reshape(1, H // 2)
    w2ee, w2eo = W2e[0::2, :], W2e[1::2, :]
    b2e_r = b2e.reshape(1, H)
    b1c_r = b1c.reshape(1, H)
    w2c_row = W2c.reshape(1, H)
    w1nh = W1n[:D]
    w1na = W1n[D:]
    b1n_r = b1n.reshape(1, H)
    b2n_r = b2n.reshape(1, D)

    ap, bp = _tc_prep(h_p, coord_p, w1h1e, w1h1o, w1h2e, w1h2o, b1ee, b1eo)

    nslice = 4
    es = E2 // nslice
    row2d = row_p.reshape(nslice, es)
    col2d = col_p.reshape(nslice, es)
    partials = jnp.zeros((NC, N2, WS), jnp.float32)
    for s in range(nslice):
        row_s = row2d[s]
        col_s = col2d[s]
        pa, pb = _sc_gather(ap, bp, row_s.reshape(-1, CG),
                            col_s.reshape(-1, CG), es)
        ef_ext = _tc_edge(pa, pb,
                          lax.dynamic_slice_in_dim(ea_p, s * es, es),
                          wre, wro, w1ae, w1ao, w2ee, w2eo, b2e_r,
                          W1c, b1c_r, w2c_row)
        partials = _sc_scatter(ef_ext, row_s.reshape(-1, CS), partials)

    hn, cn = _tc_node(partials[0], partials[1], h_p, coord_p,
                      w1nh, w1na, b1n_r, W2n, b2n_r)
    return hn[:N], cn[:N]


# speedup vs baseline: 1.0321x; 1.0012x over previous
import functools

import jax
import jax.numpy as jnp
from jax import lax
from jax.experimental import pallas as pl
from jax.experimental.pallas import tpu as pltpu
from jax.experimental.pallas import tpu_sc as plsc

N = 10000
E = 320000
D = 128
DE = 16
H = 128

N2 = 10240
E2 = 327680
NC = 2
NS = 16
NW = NC * NS
EPW = E2 // NW
WG = 80
WS = 136
CG = 128
CS = 128


def _silu(x):
    return x * jax.nn.sigmoid(x)


def _pack_pair(even_f32, odd_f32):
    ue = jax.lax.bitcast_convert_type(even_f32, jnp.uint32)
    uo = jax.lax.bitcast_convert_type(odd_f32, jnp.uint32)
    half = jnp.uint32(0x8000)
    lo = jnp.right_shift(ue + half, jnp.uint32(16))
    hi = jnp.bitwise_and(uo + half, jnp.uint32(0xFFFF0000))
    return jax.lax.bitcast_convert_type(jnp.bitwise_or(lo, hi), jnp.float32)


def _prep_body(h_ref, cd_ref, w1h1e_ref, w1h1o_ref, w1h2e_ref, w1h2o_ref,
               b1ee_ref, b1eo_ref, a_ref, b_ref):
    hh = h_ref[...]
    ae = jnp.dot(hh, w1h1e_ref[...], preferred_element_type=jnp.float32) + b1ee_ref[...]
    ao = jnp.dot(hh, w1h1o_ref[...], preferred_element_type=jnp.float32) + b1eo_ref[...]
    be = jnp.dot(hh, w1h2e_ref[...], preferred_element_type=jnp.float32)
    bo = jnp.dot(hh, w1h2o_ref[...], preferred_element_type=jnp.float32)
    cd = cd_ref[...]
    z13 = jnp.zeros((hh.shape[0], 13), jnp.float32)
    a_ref[...] = jnp.concatenate([_pack_pair(ae, ao), cd, z13], axis=1)
    b_ref[...] = jnp.concatenate([_pack_pair(be, bo), -cd, z13], axis=1)


def _tc_prep(h_p, coord_p, w1h1e, w1h1o, w1h2e, w1h2o, b1ee, b1eo):
    nb = 2048
    grid = (N2 // nb,)
    return pl.pallas_call(
        _prep_body,
        grid=grid,
        in_specs=[
            pl.BlockSpec((nb, D), lambda i: (i, 0)),
            pl.BlockSpec((nb, 3), lambda i: (i, 0)),
            pl.BlockSpec((D, H // 2), lambda i: (0, 0)),
            pl.BlockSpec((D, H // 2), lambda i: (0, 0)),
            pl.BlockSpec((D, H // 2), lambda i: (0, 0)),
            pl.BlockSpec((D, H // 2), lambda i: (0, 0)),
            pl.BlockSpec((1, H // 2), lambda i: (0, 0)),
            pl.BlockSpec((1, H // 2), lambda i: (0, 0)),
        ],
        out_specs=[
            pl.BlockSpec((nb, WG), lambda i: (i, 0)),
            pl.BlockSpec((nb, WG), lambda i: (i, 0)),
        ],
        out_shape=[
            jax.ShapeDtypeStruct((N2, WG), jnp.float32),
            jax.ShapeDtypeStruct((N2, WG), jnp.float32),
        ],
    )(h_p, coord_p, w1h1e, w1h1o, w1h2e, w1h2o, b1ee, b1eo)


def _sc_gather(ap, bp, row2d, col2d, es):
    mesh = plsc.VectorSubcoreMesh(core_axis_name="c", subcore_axis_name="s")
    epw = es // NW
    nchunk = epw // CG

    @functools.partial(
        pl.kernel,
        mesh=mesh,
        out_type=(
            jax.ShapeDtypeStruct((es, WG), jnp.float32),
            jax.ShapeDtypeStruct((es, WG), jnp.float32),
        ),
        scratch_types=[
            pltpu.VMEM((nchunk, CG), jnp.int32),
            pltpu.VMEM((nchunk, CG), jnp.int32),
            pltpu.VMEM((4, CG, WG), jnp.float32),
            pltpu.VMEM((4, CG, WG), jnp.float32),
            pltpu.SemaphoreType.DMA,
            pltpu.SemaphoreType.DMA,
            pltpu.SemaphoreType.DMA,
            pltpu.SemaphoreType.DMA,
            pltpu.SemaphoreType.DMA,
            pltpu.SemaphoreType.DMA,
            pltpu.SemaphoreType.DMA,
            pltpu.SemaphoreType.DMA,
            pltpu.SemaphoreType.DMA,
            pltpu.SemaphoreType.DMA,
        ],
        compiler_params=pltpu.CompilerParams(use_tc_tiling_on_sc=False),
    )
    def gat(ap_hbm, bp_hbm, row_hbm, col_hbm, pa_hbm, pb_hbm,
            idxa, idxb, bufa, bufb, sga0, sga1, sga2, sga3,
            sgb0, sgb1, sgb2, sgb3, swa, swb):
        wid = lax.axis_index("s") * NC + lax.axis_index("c")
        base = wid * epw
        pltpu.sync_copy(row_hbm.at[pl.ds(wid * nchunk, nchunk)], idxa)
        pltpu.sync_copy(col_hbm.at[pl.ds(wid * nchunk, nchunk)], idxb)
        sga = (sga0, sga1, sga2, sga3)
        sgb = (sgb0, sgb1, sgb2, sgb3)
        nbuf = 4

        def issue_gather(i, b):
            pltpu.make_async_copy(ap_hbm.at[idxa.at[i]], bufa.at[b],
                                  sga[b]).start()
            pltpu.make_async_copy(bp_hbm.at[idxb.at[i]], bufb.at[b],
                                  sgb[b]).start()

        def wait_gather(i, b):
            pltpu.make_async_copy(ap_hbm.at[idxa.at[i]], bufa.at[b],
                                  sga[b]).wait()
            pltpu.make_async_copy(bp_hbm.at[idxb.at[i]], bufb.at[b],
                                  sgb[b]).wait()

        def drain_wb(b):
            pltpu.make_async_copy(bufa.at[b], pa_hbm.at[pl.ds(base, CG)],
                                  swa).wait()
            pltpu.make_async_copy(bufb.at[b], pb_hbm.at[pl.ds(base, CG)],
                                  swb).wait()

        def start_wb(i, b):
            off = base + i * CG
            pltpu.make_async_copy(bufa.at[b], pa_hbm.at[pl.ds(off, CG)],
                                  swa).start()
            pltpu.make_async_copy(bufb.at[b], pb_hbm.at[pl.ds(off, CG)],
                                  swb).start()

        for b in range(nbuf - 1):
            issue_gather(b, b)

        def body(i4, carry):
            for b in range(nbuf):
                i = i4 * nbuf + b
                abuf = (b + nbuf - 1) % nbuf

                @pl.when(i + nbuf - 1 < nchunk)
                def _():
                    if b == 0:
                        @pl.when(i4 > 0)
                        def _():
                            drain_wb(abuf)
                    else:
                        drain_wb(abuf)
                    issue_gather(i + nbuf - 1, abuf)

                wait_gather(i, b)
                start_wb(i, b)
            return carry

        lax.fori_loop(0, nchunk // nbuf, body, 0)
        for b in range(nbuf):
            drain_wb(b)

    return gat(ap, bp, row2d, col2d)


def _unpack_pair(packed_f32):
    u = jax.lax.bitcast_convert_type(packed_f32, jnp.uint32)
    even = jax.lax.bitcast_convert_type(
        jnp.left_shift(u, jnp.uint32(16)), jnp.float32)
    odd = jax.lax.bitcast_convert_type(
        jnp.bitwise_and(u, jnp.uint32(0xFFFF0000)), jnp.float32)
    return even, odd


def _edge_body(pa_ref, pb_ref, ea_ref, wre_ref, wro_ref, w1ae_ref, w1ao_ref,
               w2ee_ref, w2eo_ref, b2e_ref, w1c_ref, b1c_ref, w2c_ref, out_ref):
    pa = pa_ref[...]
    pb = pb_ref[...]
    ae, ao = _unpack_pair(pa[:, :H // 2])
    be, bo = _unpack_pair(pb[:, :H // 2])
    cd = pa[:, H // 2:H // 2 + 3] + pb[:, H // 2:H // 2 + 3]
    radial = jnp.sum(cd * cd, axis=1, keepdims=True)
    ea = ea_ref[...]
    ze = ae + be + radial * wre_ref[...] + jnp.dot(
        ea, w1ae_ref[...], preferred_element_type=jnp.float32)
    zo = ao + bo + radial * wro_ref[...] + jnp.dot(
        ea, w1ao_ref[...], preferred_element_type=jnp.float32)
    ze = _silu(ze)
    zo = _silu(zo)
    ef = (jnp.dot(ze, w2ee_ref[...], preferred_element_type=jnp.float32)
          + jnp.dot(zo, w2eo_ref[...], preferred_element_type=jnp.float32)
          + b2e_ref[...])
    ef = _silu(ef)
    c1 = jnp.dot(ef, w1c_ref[...], preferred_element_type=jnp.float32) + b1c_ref[...]
    c1 = _silu(c1)
    cm = jnp.sum(c1 * w2c_ref[...], axis=1, keepdims=True)
    trans = jnp.clip(cd * cm, -100.0, 100.0)
    ones = jnp.ones((trans.shape[0], 1), jnp.float32)
    z4 = jnp.zeros((trans.shape[0], 4), jnp.float32)
    out_ref[...] = jnp.concatenate([ef, trans, ones, z4], axis=1)


def _tc_edge(pa, pb, ea_p, wre, wro, w1ae, w1ao, w2ee, w2eo, b2e, w1c, b1c,
             w2c_row):
    eb = 2048
    es = pa.shape[0]
    grid = (es // eb,)
    return pl.pallas_call(
        _edge_body,
        grid=grid,
        in_specs=[
            pl.BlockSpec((eb, WG), lambda i: (i, 0)),
            pl.BlockSpec((eb, WG), lambda i: (i, 0)),
            pl.BlockSpec((eb, DE), lambda i: (i, 0)),
            pl.BlockSpec((1, H // 2), lambda i: (0, 0)),
            pl.BlockSpec((1, H // 2), lambda i: (0, 0)),
            pl.BlockSpec((DE, H // 2), lambda i: (0, 0)),
            pl.BlockSpec((DE, H // 2), lambda i: (0, 0)),
            pl.BlockSpec((H // 2, H), lambda i: (0, 0)),
            pl.BlockSpec((H // 2, H), lambda i: (0, 0)),
            pl.BlockSpec((1, H), lambda i: (0, 0)),
            pl.BlockSpec((H, H), lambda i: (0, 0)),
            pl.BlockSpec((1, H), lambda i: (0, 0)),
            pl.BlockSpec((1, H), lambda i: (0, 0)),
        ],
        out_specs=pl.BlockSpec((eb, WS), lambda i: (i, 0)),
        out_shape=jax.ShapeDtypeStruct((es, WS), jnp.float32),
    )(pa, pb, ea_p, wre, wro, w1ae, w1ao, w2ee, w2eo, b2e, w1c, b1c, w2c_row)


def _sc_scatter(ef_ext, row2d, init_partials):
    mesh = plsc.VectorSubcoreMesh(core_axis_name="c", subcore_axis_name="s")
    es = ef_ext.shape[0]
    epw = es // NW
    nchunk = epw // CS

    @functools.partial(
        pl.kernel,
        mesh=mesh,
        out_type=jax.ShapeDtypeStruct((NC, N2, WS), jnp.float32),
        scratch_types=[
            pltpu.VMEM((nchunk, CS), jnp.int32),
            pltpu.VMEM((2, CS, WS), jnp.float32),
            pltpu.VMEM_SHARED((N2, WS), jnp.float32),
            pltpu.SemaphoreType.DMA,
            pltpu.SemaphoreType.DMA,
        ],
        compiler_params=pltpu.CompilerParams(use_tc_tiling_on_sc=False),
    )
    def sca(ef_hbm, row_hbm, init_hbm, out_hbm, idx2d, efv, acc, sl0, sl1):
        cid = lax.axis_index("c")
        sid = lax.axis_index("s")
        wid = sid * NC + cid

        @pl.when(sid == 0)
        def _():
            pltpu.sync_copy(init_hbm.at[cid], acc)

        base = wid * epw
        pltpu.sync_copy(row_hbm.at[pl.ds(wid * nchunk, nchunk)], idx2d)
        plsc.subcore_barrier()
        sl = (sl0, sl1)

        def start_load(i, b):
            pltpu.make_async_copy(ef_hbm.at[pl.ds(base + i * CS, CS)],
                                  efv.at[b], sl[b]).start()

        def wait_load(b):
            pltpu.make_async_copy(ef_hbm.at[pl.ds(base, CS)], efv.at[b],
                                  sl[b]).wait()

        start_load(0, 0)

        def body(i2, carry):
            for b in range(2):
                i = i2 * 2 + b
                if b == 0:
                    start_load(i + 1, 1)
                else:
                    @pl.when(i2 < nchunk // 2 - 1)
                    def _():
                        start_load(i + 1, 0)
                wait_load(b)
                pltpu.sync_copy(efv.at[b], acc.at[idx2d.at[i]], add=True)
            return carry

        lax.fori_loop(0, nchunk // 2, body, 0)
        plsc.subcore_barrier()

        @pl.when(sid == 0)
        def _():
            pltpu.sync_copy(acc, out_hbm.at[cid])

    return sca(ef_ext, row2d, init_partials)


def _node_body(p0_ref, p1_ref, h_ref, cd_ref, w1nh_ref, w1na_ref, b1n_ref,
               w2n_ref, b2n_ref, hn_ref, cn_ref):
    p = p0_ref[...] + p1_ref[...]
    agg = p[:, :H]
    num = p[:, H:H + 3]
    cnt = p[:, H + 3:H + 4]
    f = num / jnp.maximum(cnt, 1.0)
    cn_ref[...] = cd_ref[...] + f
    z = (jnp.dot(h_ref[...], w1nh_ref[...], preferred_element_type=jnp.float32)
         + jnp.dot(agg, w1na_ref[...], preferred_element_type=jnp.float32)
         + b1n_ref[...])
    z = _silu(z)
    hn_ref[...] = h_ref[...] + jnp.dot(
        z, w2n_ref[...], preferred_element_type=jnp.float32) + b2n_ref[...]


def _tc_node(p0, p1, h_p, coord_p, w1nh, w1na, b1n, w2n, b2n):
    nb = 2048
    grid = (N2 // nb,)
    return pl.pallas_call(
        _node_body,
        grid=grid,
        in_specs=[
            pl.BlockSpec((nb, WS), lambda i: (i, 0)),
            pl.BlockSpec((nb, WS), lambda i: (i, 0)),
            pl.BlockSpec((nb, D), lambda i: (i, 0)),
            pl.BlockSpec((nb, 3), lambda i: (i, 0)),
            pl.BlockSpec((D, H), lambda i: (0, 0)),
            pl.BlockSpec((H, H), lambda i: (0, 0)),
            pl.BlockSpec((1, H), lambda i: (0, 0)),
            pl.BlockSpec((H, D), lambda i: (0, 0)),
            pl.BlockSpec((1, D), lambda i: (0, 0)),
        ],
        out_specs=[
            pl.BlockSpec((nb, D), lambda i: (i, 0)),
            pl.BlockSpec((nb, 3), lambda i: (i, 0)),
        ],
        out_shape=[
            jax.ShapeDtypeStruct((N2, D), jnp.float32),
            jax.ShapeDtypeStruct((N2, 3), jnp.float32),
        ],
    )(p0, p1, h_p, coord_p, w1nh, w1na, b1n, w2n, b2n)


@jax.jit
def kernel(h, coord, edge_index, edge_attr,
           W1e, b1e, W2e, b2e, W1n, b1n, W2n, b2n, W1c, b1c, W2c):
    row = edge_index[0].astype(jnp.int32)
    col = edge_index[1].astype(jnp.int32)

    h_p = jnp.zeros((N2, D), jnp.float32).at[:N].set(h)
    coord_p = jnp.zeros((N2, 3), jnp.float32).at[:N].set(coord)
    row_p = jnp.full((E2,), N, jnp.int32).at[:E].set(row)
    col_p = jnp.zeros((E2,), jnp.int32).at[:E].set(col)
    ea_p = jnp.zeros((E2, DE), jnp.float32).at[:E].set(edge_attr)

    w1h1 = W1e[0:D]
    w1h2 = W1e[D:2 * D]
    wr = W1e[2 * D:2 * D + 1]
    w1a = W1e[2 * D + 1:]
    w1h1e, w1h1o = w1h1[:, 0::2], w1h1[:, 1::2]
    w1h2e, w1h2o = w1h2[:, 0::2], w1h2[:, 1::2]
    wre, wro = wr[:, 0::2], wr[:, 1::2]
    w1ae, w1ao = w1a[:, 0::2], w1a[:, 1::2]
    b1ee = b1e[0::2].reshape(1, H // 2)
    b1eo = b1e[1::2].reshape(1, H // 2)
    w2ee, w2eo = W2e[0::2, :], W2e[1::2, :]
    b2e_r = b2e.reshape(1, H)
    b1c_r = b1c.reshape(1, H)
    w2c_row = W2c.reshape(1, H)
    w1nh = W1n[:D]
    w1na = W1n[D:]
    b1n_r = b1n.reshape(1, H)
    b2n_r = b2n.reshape(1, D)

    ap, bp = _tc_prep(h_p, coord_p, w1h1e, w1h1o, w1h2e, w1h2o, b1ee, b1eo)

    nslice = 4
    es = E2 // nslice
    row2d = row_p.reshape(nslice, es)
    col2d = col_p.reshape(nslice, es)
    partials = jnp.zeros((NC, N2, WS), jnp.float32)
    for s in range(nslice):
        row_s = row2d[s]
        col_s = col2d[s]
        pa, pb = _sc_gather(ap, bp, row_s.reshape(-1, CG),
                            col_s.reshape(-1, CG), es)
        ef_ext = _tc_edge(pa, pb,
                          lax.dynamic_slice_in_dim(ea_p, s * es, es),
                          wre, wro, w1ae, w1ao, w2ee, w2eo, b2e_r,
                          W1c, b1c_r, w2c_row)
        partials = _sc_scatter(ef_ext, row_s.reshape(-1, CS), partials)

    hn, cn = _tc_node(partials[0], partials[1], h_p, coord_p,
                      w1nh, w1na, b1n_r, W2n, b2n_r)
    return hn[:N], cn[:N]
